# Initial kernel scaffold; baseline (speedup 1.0000x reference)
import jax
import jax.numpy as jnp
from jax.experimental import pallas as pl


def kernel(x, emb):
    raise NotImplementedError("write your pallas kernel here")


# speedup vs baseline: 3.0207x; 3.0207x over previous
import jax
import jax.numpy as jnp
from jax.experimental import pallas as pl

_DIM = 1024
_SCALE = _DIM ** (-0.5)
_BLOCK_ROWS = 1024


def _scale_kernel(emb_ref, out_ref):
    out_ref[...] = emb_ref[...] * _SCALE


def kernel(x, emb):
    seq_len = x.shape[1]
    rows = emb.shape[0]
    assert seq_len == rows
    grid = rows // _BLOCK_ROWS
    return pl.pallas_call(
        _scale_kernel,
        grid=(grid,),
        in_specs=[pl.BlockSpec((_BLOCK_ROWS, _DIM), lambda i: (i, 0))],
        out_specs=pl.BlockSpec((_BLOCK_ROWS, _DIM), lambda i: (i, 0)),
        out_shape=jax.ShapeDtypeStruct((rows, _DIM), emb.dtype),
    )(emb)
